# baseline (speedup 1.0000x reference)
import functools

import jax
import jax.numpy as jnp
from jax import lax
from jax.experimental import pallas as pl
from jax.experimental.pallas import tpu as pltpu
from jax.experimental.pallas import tpu_sc as plsc

_NWORDS = 1000000
_EMB = 64
_NIDX = 16384 * 50

_NC, _NS, _NL = 2, 16, 16
_NW = _NC * _NS
_BPW = _NIDX // _NW

_BC = 16384
_CT = 10240
_SSPLIT = _NW * _CT
_TCCOLS = _NWORDS - _SSPLIT
_NBLK = (_TCCOLS + _BC - 1) // _BC
_SCCH = 512
_NCH = _CT // _SCCH


def _rowsum_body(x_ref, o_ref):
    o_ref[...] = jnp.sum(x_ref[...], axis=0)


_rowsum_call = pl.pallas_call(
    _rowsum_body,
    grid=(_NBLK,),
    in_specs=[pl.BlockSpec((_EMB, _BC), lambda i: (0, i + _SSPLIT // _BC))],
    out_specs=pl.BlockSpec((_BC,), lambda i: (i,)),
    out_shape=jax.ShapeDtypeStruct((_TCCOLS,), jnp.float32),
)


@functools.partial(
    pl.kernel,
    mesh=plsc.VectorSubcoreMesh(core_axis_name="c", subcore_axis_name="s"),
    out_type=jax.ShapeDtypeStruct((_SSPLIT,), jnp.float32),
    scratch_types=[
        pltpu.VMEM((_EMB, _SCCH), jnp.float32),
        pltpu.VMEM((_EMB, _SCCH), jnp.float32),
        pltpu.VMEM((_CT,), jnp.float32),
        pltpu.SemaphoreType.DMA,
        pltpu.SemaphoreType.DMA,
    ],
)
def _sc_rowsum(table_hbm, rs_hbm, buf0, buf1, rs_v, sem0, sem1):
    wid = lax.axis_index("s") * _NC + lax.axis_index("c")
    base = wid * _CT
    bufs = (buf0, buf1)
    sems = (sem0, sem1)

    pltpu.async_copy(table_hbm.at[:, pl.ds(base, _SCCH)], buf0, sem0)

    for c in range(_NCH):
        buf = bufs[c % 2]
        sem = sems[c % 2]
        pltpu.make_async_copy(
            table_hbm.at[:, pl.ds(base + c * _SCCH, _SCCH)], buf, sem
        ).wait()
        if c + 1 < _NCH:
            pltpu.async_copy(
                table_hbm.at[:, pl.ds(base + (c + 1) * _SCCH, _SCCH)],
                bufs[(c + 1) % 2],
                sems[(c + 1) % 2],
            )

        def col_body(j, _):
            zero = jnp.zeros((_NL,), jnp.float32)
            a0, a1, a2, a3 = zero, zero, zero, zero
            for r in range(0, _EMB, 4):
                a0 = a0 + buf[r, pl.ds(j * _NL, _NL)]
                a1 = a1 + buf[r + 1, pl.ds(j * _NL, _NL)]
                a2 = a2 + buf[r + 2, pl.ds(j * _NL, _NL)]
                a3 = a3 + buf[r + 3, pl.ds(j * _NL, _NL)]
            rs_v[pl.ds(c * _SCCH + j * _NL, _NL)] = (a0 + a1) + (a2 + a3)
            return 0

        lax.fori_loop(0, _SCCH // _NL, col_body, 0)

    pltpu.sync_copy(rs_v, rs_hbm.at[pl.ds(base, _CT)])


@functools.partial(
    pl.kernel,
    mesh=plsc.VectorSubcoreMesh(core_axis_name="c", subcore_axis_name="s"),
    out_type=jax.ShapeDtypeStruct((_NW, _NL), jnp.float32),
    scratch_types=[
        pltpu.VMEM((_BPW,), jnp.int32),
        pltpu.VMEM((_BPW,), jnp.float32),
        pltpu.VMEM((_NL,), jnp.float32),
        pltpu.SemaphoreType.DMA,
    ],
)
def _sc_gather_sum(words_hbm, rowsums_hbm, out_hbm, idx_v, vals_v, acc_v, sem):
    wid = lax.axis_index("s") * _NC + lax.axis_index("c")
    base = wid * _BPW
    pltpu.sync_copy(words_hbm.at[pl.ds(base, _BPW)], idx_v)
    pltpu.async_copy(rowsums_hbm.at[idx_v], vals_v, sem).wait()

    zero = jnp.zeros((_NL,), jnp.float32)

    def body(i, accs):
        a0, a1, a2, a3 = accs
        b = i * 4 * _NL
        a0 = a0 + vals_v[pl.ds(b, _NL)]
        a1 = a1 + vals_v[pl.ds(b + _NL, _NL)]
        a2 = a2 + vals_v[pl.ds(b + 2 * _NL, _NL)]
        a3 = a3 + vals_v[pl.ds(b + 3 * _NL, _NL)]
        return (a0, a1, a2, a3)

    a0, a1, a2, a3 = lax.fori_loop(
        0, _BPW // (4 * _NL), body, (zero, zero, zero, zero)
    )
    acc_v[...] = (a0 + a1) + (a2 + a3)
    pltpu.sync_copy(acc_v, out_hbm.at[wid])


def _mlp_body(p_ref, w1_ref, b1_ref, w2_ref, b2_ref, wo_ref, bo_ref, o_ref):
    s = jnp.sum(p_ref[...])
    h1 = jnp.tanh(s * w1_ref[...] + b1_ref[...])
    h2 = jnp.tanh(
        jnp.dot(h1, w2_ref[...], preferred_element_type=jnp.float32) + b2_ref[...]
    )
    o_ref[...] = (
        jnp.dot(h2, wo_ref[...], preferred_element_type=jnp.float32) + bo_ref[...]
    )


def _mlp_call(partials, W1, b1, W2, b2, Wout, bout):
    return pl.pallas_call(
        _mlp_body,
        out_shape=jax.ShapeDtypeStruct((1, bout.shape[-1]), jnp.float32),
    )(partials, W1, b1, W2, b2, Wout, bout)


def kernel(words, table, W1, b1, W2, b2, Wout, bout):
    words_flat = words.T.reshape(-1).astype(jnp.int32)
    table_t = table.T
    rs_head = _sc_rowsum(table_t)
    rs_tail = _rowsum_call(table_t)
    rowsums = jnp.concatenate([rs_head, rs_tail])
    partials = _sc_gather_sum(words_flat, rowsums)
    return _mlp_call(
        partials,
        W1,
        b1.reshape(1, -1),
        W2,
        b2.reshape(1, -1),
        Wout,
        bout.reshape(1, -1),
    )


# speedup vs baseline: 1.0144x; 1.0144x over previous
import functools

import jax
import jax.numpy as jnp
from jax import lax
from jax.experimental import pallas as pl
from jax.experimental.pallas import tpu as pltpu
from jax.experimental.pallas import tpu_sc as plsc

_NWORDS = 1000000
_EMB = 64
_NIDX = 16384 * 50

_NC, _NS, _NL = 2, 16, 16
_NW = _NC * _NS
_BPW = _NIDX // _NW

_BC = 16384
_NBLK = (_NWORDS + _BC - 1) // _BC


def _rowsum_body(x_ref, o_ref):
    o_ref[...] = jnp.sum(x_ref[...], axis=0)


_rowsum_call = pl.pallas_call(
    _rowsum_body,
    grid=(_NBLK,),
    in_specs=[pl.BlockSpec((_EMB, _BC), lambda i: (0, i))],
    out_specs=pl.BlockSpec((_BC,), lambda i: (i,)),
    out_shape=jax.ShapeDtypeStruct((_NWORDS,), jnp.float32),
)


@functools.partial(
    pl.kernel,
    mesh=plsc.VectorSubcoreMesh(core_axis_name="c", subcore_axis_name="s"),
    out_type=jax.ShapeDtypeStruct((_NW, _NL), jnp.float32),
    scratch_types=[
        pltpu.VMEM((_BPW,), jnp.int32),
        pltpu.VMEM((_BPW,), jnp.float32),
        pltpu.VMEM((_NL,), jnp.float32),
        pltpu.SemaphoreType.DMA,
    ],
)
def _sc_gather_sum(words_hbm, rowsums_hbm, out_hbm, idx_v, vals_v, acc_v, sem):
    wid = lax.axis_index("s") * _NC + lax.axis_index("c")
    base = wid * _BPW
    nch = 4
    ch = _BPW // nch

    def stage(c):
        pltpu.sync_copy(
            words_hbm.at[pl.ds(base + c * ch, ch)], idx_v.at[pl.ds(c * ch, ch)]
        )

    def gather(c):
        return pltpu.async_copy(
            rowsums_hbm.at[idx_v.at[pl.ds(c * ch, ch)]],
            vals_v.at[pl.ds(c * ch, ch)],
            sem,
        )

    stage(0)
    handles = [gather(0)]
    zero = jnp.zeros((_NL,), jnp.float32)
    accs = (zero, zero, zero, zero)
    for c in range(nch):
        if c + 1 < nch:
            stage(c + 1)
            handles.append(gather(c + 1))
        handles[c].wait()

        def body(i, a, c=c):
            a0, a1, a2, a3 = a
            b = c * ch + i * 4 * _NL
            a0 = a0 + vals_v[pl.ds(b, _NL)]
            a1 = a1 + vals_v[pl.ds(b + _NL, _NL)]
            a2 = a2 + vals_v[pl.ds(b + 2 * _NL, _NL)]
            a3 = a3 + vals_v[pl.ds(b + 3 * _NL, _NL)]
            return (a0, a1, a2, a3)

        accs = lax.fori_loop(0, ch // (4 * _NL), body, accs)

    a0, a1, a2, a3 = accs
    acc_v[...] = (a0 + a1) + (a2 + a3)
    pltpu.sync_copy(acc_v, out_hbm.at[wid])


def _mlp_body(p_ref, w1_ref, b1_ref, w2_ref, b2_ref, wo_ref, bo_ref, o_ref):
    s = jnp.sum(p_ref[...])
    h1 = jnp.tanh(s * w1_ref[...] + b1_ref[...])
    h2 = jnp.tanh(
        jnp.dot(h1, w2_ref[...], preferred_element_type=jnp.float32) + b2_ref[...]
    )
    o_ref[...] = (
        jnp.dot(h2, wo_ref[...], preferred_element_type=jnp.float32) + bo_ref[...]
    )


def _mlp_call(partials, W1, b1, W2, b2, Wout, bout):
    return pl.pallas_call(
        _mlp_body,
        out_shape=jax.ShapeDtypeStruct((1, bout.shape[-1]), jnp.float32),
    )(partials, W1, b1, W2, b2, Wout, bout)


def kernel(words, table, W1, b1, W2, b2, Wout, bout):
    words_flat = words.T.reshape(-1).astype(jnp.int32)
    rowsums = _rowsum_call(table.T)
    partials = _sc_gather_sum(words_flat, rowsums)
    return _mlp_call(
        partials,
        W1,
        b1.reshape(1, -1),
        W2,
        b2.reshape(1, -1),
        Wout,
        bout.reshape(1, -1),
    )


# speedup vs baseline: 1.0170x; 1.0026x over previous
import functools

import jax
import jax.numpy as jnp
from jax import lax
from jax.experimental import pallas as pl
from jax.experimental.pallas import tpu as pltpu
from jax.experimental.pallas import tpu_sc as plsc

_NWORDS = 1000000
_EMB = 64
_NIDX = 16384 * 50

_NC, _NS, _NL = 2, 16, 16
_NW = _NC * _NS
_BPW = _NIDX // _NW

_BC = 16384
_NBLK = (_NWORDS + _BC - 1) // _BC


def _rowsum_body(x_ref, o_ref):
    o_ref[...] = jnp.sum(x_ref[...], axis=0)


_rowsum_call = pl.pallas_call(
    _rowsum_body,
    grid=(_NBLK,),
    in_specs=[pl.BlockSpec((_EMB, _BC), lambda i: (0, i))],
    out_specs=pl.BlockSpec((_BC,), lambda i: (i,)),
    out_shape=jax.ShapeDtypeStruct((_NWORDS,), jnp.float32),
)


@functools.partial(
    pl.kernel,
    mesh=plsc.VectorSubcoreMesh(core_axis_name="c", subcore_axis_name="s"),
    out_type=jax.ShapeDtypeStruct((_NW, _NL), jnp.float32),
    scratch_types=[
        pltpu.VMEM((_BPW,), jnp.int32),
        pltpu.VMEM((_BPW,), jnp.float32),
        pltpu.VMEM((_NL,), jnp.float32),
        pltpu.SemaphoreType.DMA,
    ],
)
def _sc_gather_sum(words_hbm, rowsums_hbm, out_hbm, idx_v, vals_v, acc_v, sem):
    wid = lax.axis_index("s") * _NC + lax.axis_index("c")
    base = wid * _BPW
    pltpu.sync_copy(words_hbm.at[pl.ds(base, _BPW)], idx_v)
    pltpu.async_copy(rowsums_hbm.at[idx_v], vals_v, sem).wait()

    zero = jnp.zeros((_NL,), jnp.float32)

    def body(i, accs):
        a0, a1, a2, a3 = accs
        b = i * 4 * _NL
        a0 = a0 + vals_v[pl.ds(b, _NL)]
        a1 = a1 + vals_v[pl.ds(b + _NL, _NL)]
        a2 = a2 + vals_v[pl.ds(b + 2 * _NL, _NL)]
        a3 = a3 + vals_v[pl.ds(b + 3 * _NL, _NL)]
        return (a0, a1, a2, a3)

    a0, a1, a2, a3 = lax.fori_loop(
        0, _BPW // (4 * _NL), body, (zero, zero, zero, zero)
    )
    acc_v[...] = (a0 + a1) + (a2 + a3)
    pltpu.sync_copy(acc_v, out_hbm.at[wid])


def _mlp_body(p_ref, w1_ref, b1_ref, w2_ref, b2_ref, wo_ref, bo_ref, o_ref):
    s = jnp.sum(p_ref[...])
    h1 = jnp.tanh(s * w1_ref[...] + b1_ref[...])
    h2 = jnp.tanh(
        jnp.dot(h1, w2_ref[...], preferred_element_type=jnp.float32) + b2_ref[...]
    )
    o_ref[...] = (
        jnp.dot(h2, wo_ref[...], preferred_element_type=jnp.float32) + bo_ref[...]
    )


def _mlp_call(partials, W1, b1, W2, b2, Wout, bout):
    return pl.pallas_call(
        _mlp_body,
        out_shape=jax.ShapeDtypeStruct((1, bout.shape[-1]), jnp.float32),
    )(partials, W1, b1, W2, b2, Wout, bout)


def kernel(words, table, W1, b1, W2, b2, Wout, bout):
    words_flat = words.T.reshape(-1).astype(jnp.int32)
    rowsums = _rowsum_call(table.T)
    partials = _sc_gather_sum(words_flat, rowsums)
    return _mlp_call(
        partials,
        W1,
        b1.reshape(1, -1),
        W2,
        b2.reshape(1, -1),
        Wout,
        bout.reshape(1, -1),
    )


# speedup vs baseline: 1.0768x; 1.0588x over previous
import functools

import jax
import jax.numpy as jnp
from jax import lax
from jax.experimental import pallas as pl
from jax.experimental.pallas import tpu as pltpu
from jax.experimental.pallas import tpu_sc as plsc

_NWORDS = 1000000
_EMB = 64
_NIDX = 16384 * 50

_NC, _NS, _NL = 2, 16, 16
_NW = _NC * _NS
_BPW = _NIDX // _NW

_HB = 1048576
_TSL = _HB // _NS
_ZCH = 8192
_ICH = 5120
_DCH = 8192

_BC = 16384
_NBLK = _HB // _BC


def _rowsum_body(x_ref, o_ref):
    o_ref[...] = jnp.sum(x_ref[...], axis=0)


_rowsum_call = pl.pallas_call(
    _rowsum_body,
    grid=(_NBLK,),
    in_specs=[pl.BlockSpec((_EMB, _BC), lambda i: (0, jnp.minimum(i, 61)))],
    out_specs=pl.BlockSpec((_BC,), lambda i: (i,)),
    out_shape=jax.ShapeDtypeStruct((_HB,), jnp.float32),
)


@functools.partial(
    pl.kernel,
    mesh=plsc.VectorSubcoreMesh(core_axis_name="c", subcore_axis_name="s"),
    out_type=jax.ShapeDtypeStruct((_NC, _HB), jnp.float32),
    scratch_types=[
        pltpu.VMEM((_ICH,), jnp.int32),
        pltpu.VMEM((_ICH,), jnp.float32),
        pltpu.VMEM((_ZCH,), jnp.float32),
        pltpu.VMEM_SHARED((_HB,), jnp.float32),
    ],
)
def _sc_hist(words_hbm, hist_hbm, idx_v, ones_v, zb_v, bins_sh):
    cid = lax.axis_index("c")
    sid = lax.axis_index("s")
    wid = sid * _NC + cid
    base = wid * _BPW

    one = jnp.full((_NL,), 1.0, jnp.float32)
    zero = jnp.zeros((_NL,), jnp.float32)

    def fill_ones(i, _):
        ones_v[pl.ds(i * _NL, _NL)] = one
        return 0

    lax.fori_loop(0, _ICH // _NL, fill_ones, 0)

    def fill_zero(i, _):
        zb_v[pl.ds(i * _NL, _NL)] = zero
        return 0

    lax.fori_loop(0, _ZCH // _NL, fill_zero, 0)

    for k in range(_TSL // _ZCH):
        pltpu.sync_copy(zb_v, bins_sh.at[pl.ds(sid * _TSL + k * _ZCH, _ZCH)])
    plsc.subcore_barrier()

    for c in range(_BPW // _ICH):
        pltpu.sync_copy(words_hbm.at[pl.ds(base + c * _ICH, _ICH)], idx_v)
        pltpu.sync_copy(ones_v, bins_sh.at[idx_v], add=True)
    plsc.subcore_barrier()

    for k in range(_TSL // _ZCH):
        off = sid * _TSL + k * _ZCH
        pltpu.sync_copy(bins_sh.at[pl.ds(off, _ZCH)], zb_v)
        pltpu.sync_copy(zb_v, hist_hbm.at[cid, pl.ds(off, _ZCH)])


@functools.partial(
    pl.kernel,
    mesh=plsc.VectorSubcoreMesh(core_axis_name="c", subcore_axis_name="s"),
    out_type=jax.ShapeDtypeStruct((_NW, _NL), jnp.float32),
    scratch_types=[
        pltpu.VMEM((_DCH,), jnp.float32),
        pltpu.VMEM((_DCH,), jnp.float32),
        pltpu.VMEM((_NL,), jnp.float32),
    ],
)
def _sc_dot(hist_hbm, rowsums_hbm, out_hbm, rbuf, cbuf, acc_v):
    cid = lax.axis_index("c")
    sid = lax.axis_index("s")
    wid = sid * _NC + cid

    zero = jnp.zeros((_NL,), jnp.float32)
    accs = (zero, zero, zero, zero)
    for ch in range(_TSL // _DCH):
        off = sid * _TSL + ch * _DCH
        pltpu.sync_copy(rowsums_hbm.at[pl.ds(off, _DCH)], rbuf)
        pltpu.sync_copy(hist_hbm.at[cid, pl.ds(off, _DCH)], cbuf)

        def body(i, a):
            a0, a1, a2, a3 = a
            b = i * 4 * _NL
            a0 = a0 + rbuf[pl.ds(b, _NL)] * cbuf[pl.ds(b, _NL)]
            a1 = a1 + rbuf[pl.ds(b + _NL, _NL)] * cbuf[pl.ds(b + _NL, _NL)]
            a2 = a2 + rbuf[pl.ds(b + 2 * _NL, _NL)] * cbuf[pl.ds(b + 2 * _NL, _NL)]
            a3 = a3 + rbuf[pl.ds(b + 3 * _NL, _NL)] * cbuf[pl.ds(b + 3 * _NL, _NL)]
            return (a0, a1, a2, a3)

        accs = lax.fori_loop(0, _DCH // (4 * _NL), body, accs)

    a0, a1, a2, a3 = accs
    acc_v[...] = (a0 + a1) + (a2 + a3)
    pltpu.sync_copy(acc_v, out_hbm.at[wid])


def _mlp_body(p_ref, w1_ref, b1_ref, w2_ref, b2_ref, wo_ref, bo_ref, o_ref):
    s = jnp.sum(p_ref[...])
    h1 = jnp.tanh(s * w1_ref[...] + b1_ref[...])
    h2 = jnp.tanh(
        jnp.dot(h1, w2_ref[...], preferred_element_type=jnp.float32) + b2_ref[...]
    )
    o_ref[...] = (
        jnp.dot(h2, wo_ref[...], preferred_element_type=jnp.float32) + bo_ref[...]
    )


def _mlp_call(partials, W1, b1, W2, b2, Wout, bout):
    return pl.pallas_call(
        _mlp_body,
        out_shape=jax.ShapeDtypeStruct((1, bout.shape[-1]), jnp.float32),
    )(partials, W1, b1, W2, b2, Wout, bout)


def kernel(words, table, W1, b1, W2, b2, Wout, bout):
    words_flat = words.T.reshape(-1).astype(jnp.int32)
    hist = _sc_hist(words_flat)
    rowsums = _rowsum_call(table.T)
    partials = _sc_dot(hist, rowsums)
    return _mlp_call(
        partials,
        W1,
        b1.reshape(1, -1),
        W2,
        b2.reshape(1, -1),
        Wout,
        bout.reshape(1, -1),
    )


# speedup vs baseline: 1.1823x; 1.0980x over previous
import functools

import jax
import jax.numpy as jnp
from jax import lax
from jax.experimental import pallas as pl
from jax.experimental.pallas import tpu as pltpu
from jax.experimental.pallas import tpu_sc as plsc

_NWORDS = 1000000
_EMB = 64
_NIDX = 16384 * 50

_NC, _NS, _NL = 2, 16, 16
_NW = _NC * _NS
_BPW = _NIDX // _NW

_HB = 1048576
_TSL = _HB // _NS
_ZCH = 8192
_ICH = 5120

_BC = 16384
_NBLK = (_NWORDS + _BC - 1) // _BC
_NRS = _NBLK * _BC
_DSL = _NRS // _NS
_DCH = 15872


def _rowsum_body(x_ref, o_ref):
    o_ref[...] = jnp.sum(x_ref[...], axis=0)


_rowsum_call = pl.pallas_call(
    _rowsum_body,
    grid=(_NBLK,),
    in_specs=[pl.BlockSpec((_EMB, _BC), lambda i: (0, i))],
    out_specs=pl.BlockSpec((_BC,), lambda i: (i,)),
    out_shape=jax.ShapeDtypeStruct((_NRS,), jnp.float32),
)


@functools.partial(
    pl.kernel,
    mesh=plsc.VectorSubcoreMesh(core_axis_name="c", subcore_axis_name="s"),
    out_type=jax.ShapeDtypeStruct((_NC, _HB), jnp.float32),
    scratch_types=[
        pltpu.VMEM((_ICH,), jnp.int32),
        pltpu.VMEM((_ICH,), jnp.float32),
        pltpu.VMEM((_ZCH,), jnp.float32),
        pltpu.VMEM_SHARED((_HB,), jnp.float32),
    ],
)
def _sc_hist(words_hbm, hist_hbm, idx_v, ones_v, zb_v, bins_sh):
    cid = lax.axis_index("c")
    sid = lax.axis_index("s")
    wid = sid * _NC + cid
    base = wid * _BPW

    one = jnp.full((_NL,), 1.0, jnp.float32)
    zero = jnp.zeros((_NL,), jnp.float32)

    def fill_ones(i, _):
        ones_v[pl.ds(i * _NL, _NL)] = one
        return 0

    lax.fori_loop(0, _ICH // _NL, fill_ones, 0)

    def fill_zero(i, _):
        zb_v[pl.ds(i * _NL, _NL)] = zero
        return 0

    lax.fori_loop(0, _ZCH // _NL, fill_zero, 0)

    for k in range(_TSL // _ZCH):
        pltpu.sync_copy(zb_v, bins_sh.at[pl.ds(sid * _TSL + k * _ZCH, _ZCH)])
    plsc.subcore_barrier()

    for c in range(_BPW // _ICH):
        pltpu.sync_copy(words_hbm.at[pl.ds(base + c * _ICH, _ICH)], idx_v)
        pltpu.sync_copy(ones_v, bins_sh.at[idx_v], add=True)
    plsc.subcore_barrier()

    for k in range(_TSL // _ZCH):
        off = sid * _TSL + k * _ZCH
        pltpu.sync_copy(bins_sh.at[pl.ds(off, _ZCH)], zb_v)
        pltpu.sync_copy(zb_v, hist_hbm.at[cid, pl.ds(off, _ZCH)])


@functools.partial(
    pl.kernel,
    mesh=plsc.VectorSubcoreMesh(core_axis_name="c", subcore_axis_name="s"),
    out_type=jax.ShapeDtypeStruct((_NW, _NL), jnp.float32),
    scratch_types=[
        pltpu.VMEM((_DCH,), jnp.float32),
        pltpu.VMEM((_DCH,), jnp.float32),
        pltpu.VMEM((_DCH,), jnp.float32),
        pltpu.VMEM((_DCH,), jnp.float32),
        pltpu.VMEM((_NL,), jnp.float32),
        pltpu.SemaphoreType.DMA,
        pltpu.SemaphoreType.DMA,
    ],
)
def _sc_dot(hist_hbm, rowsums_hbm, out_hbm, rb0, cb0, rb1, cb1, acc_v, sem0, sem1):
    cid = lax.axis_index("c")
    sid = lax.axis_index("s")
    wid = sid * _NC + cid
    bufs = ((rb0, cb0, sem0), (rb1, cb1, sem1))
    nch = _DSL // _DCH

    def start(ch):
        off = sid * _DSL + ch * _DCH
        r, c, s = bufs[ch % 2]
        return (
            pltpu.async_copy(rowsums_hbm.at[pl.ds(off, _DCH)], r, s),
            pltpu.async_copy(hist_hbm.at[cid, pl.ds(off, _DCH)], c, s),
        )

    handles = [start(0)]
    zero = jnp.zeros((_NL,), jnp.float32)
    accs = (zero, zero, zero, zero)
    for ch in range(nch):
        if ch + 1 < nch:
            handles.append(start(ch + 1))
        h1, h2 = handles[ch]
        h1.wait()
        h2.wait()
        rbuf, cbuf, _ = bufs[ch % 2]

        def body(i, a, rbuf=rbuf, cbuf=cbuf):
            a0, a1, a2, a3 = a
            b = i * 4 * _NL

            def fma(acc, o):
                cnt = cbuf[pl.ds(b + o, _NL)]
                prod = jnp.where(cnt > 0.0, rbuf[pl.ds(b + o, _NL)] * cnt, 0.0)
                return acc + prod

            return (fma(a0, 0), fma(a1, _NL), fma(a2, 2 * _NL), fma(a3, 3 * _NL))

        accs = lax.fori_loop(0, _DCH // (4 * _NL), body, accs)

    a0, a1, a2, a3 = accs
    acc_v[...] = (a0 + a1) + (a2 + a3)
    pltpu.sync_copy(acc_v, out_hbm.at[wid])


def _mlp_body(p_ref, w1_ref, b1_ref, w2_ref, b2_ref, wo_ref, bo_ref, o_ref):
    s = jnp.sum(p_ref[...])
    h1 = jnp.tanh(s * w1_ref[...] + b1_ref[...])
    h2 = jnp.tanh(
        jnp.dot(h1, w2_ref[...], preferred_element_type=jnp.float32) + b2_ref[...]
    )
    o_ref[...] = (
        jnp.dot(h2, wo_ref[...], preferred_element_type=jnp.float32) + bo_ref[...]
    )


def _mlp_call(partials, W1, b1, W2, b2, Wout, bout):
    return pl.pallas_call(
        _mlp_body,
        out_shape=jax.ShapeDtypeStruct((1, bout.shape[-1]), jnp.float32),
    )(partials, W1, b1, W2, b2, Wout, bout)


def kernel(words, table, W1, b1, W2, b2, Wout, bout):
    words_flat = words.T.reshape(-1).astype(jnp.int32)
    hist = _sc_hist(words_flat)
    rowsums = _rowsum_call(table.T)
    partials = _sc_dot(hist, rowsums)
    return _mlp_call(
        partials,
        W1,
        b1.reshape(1, -1),
        W2,
        b2.reshape(1, -1),
        Wout,
        bout.reshape(1, -1),
    )


# speedup vs baseline: 1.2324x; 1.0424x over previous
import functools

import jax
import jax.numpy as jnp
from jax import lax
from jax.experimental import pallas as pl
from jax.experimental.pallas import tpu as pltpu
from jax.experimental.pallas import tpu_sc as plsc

_NWORDS = 1000000
_EMB = 64
_NIDX = 16384 * 50

_NC, _NS, _NL = 2, 16, 16
_NW = _NC * _NS
_BPW = _NIDX // _NW

_HB = 1048576
_TSL = _HB // _NS
_ZCH = 4096
_ICH = 5120

_BC = 16384
_NBLK = (_NWORDS + _BC - 1) // _BC
_NRS = _NBLK * _BC
_DSL = _NRS // _NS
_DCH = 15872


def _rowsum_body(x_ref, o_ref):
    o_ref[...] = jnp.sum(x_ref[...], axis=0)


_rowsum_call = pl.pallas_call(
    _rowsum_body,
    grid=(_NBLK,),
    in_specs=[pl.BlockSpec((_EMB, _BC), lambda i: (0, i))],
    out_specs=pl.BlockSpec((_BC,), lambda i: (i,)),
    out_shape=jax.ShapeDtypeStruct((_NRS,), jnp.float32),
)


@functools.partial(
    pl.kernel,
    mesh=plsc.VectorSubcoreMesh(core_axis_name="c", subcore_axis_name="s"),
    out_type=jax.ShapeDtypeStruct((_NC, _HB), jnp.float32),
    scratch_types=[
        pltpu.VMEM((50, 512), jnp.int32),
        pltpu.VMEM((_BPW,), jnp.int32),
        pltpu.VMEM((_ICH,), jnp.float32),
        pltpu.VMEM((_ZCH,), jnp.float32),
        pltpu.VMEM_SHARED((_HB,), jnp.float32),
    ],
)
def _sc_hist(words_hbm, hist_hbm, w2_v, idx_v, ones_v, zb_v, bins_sh):
    cid = lax.axis_index("c")
    sid = lax.axis_index("s")
    wid = sid * _NC + cid

    pltpu.sync_copy(words_hbm.at[:, pl.ds(wid * 512, 512)], w2_v)

    one = jnp.full((_NL,), 1.0, jnp.float32)
    zero = jnp.zeros((_NL,), jnp.float32)

    def repack(i, _):
        r = i >> 5
        k = (i & 31) * _NL
        idx_v[pl.ds(r * 512 + k, _NL)] = w2_v[r, pl.ds(k, _NL)]
        return 0

    lax.fori_loop(0, 50 * 32, repack, 0)

    def fill_ones(i, _):
        ones_v[pl.ds(i * _NL, _NL)] = one
        return 0

    lax.fori_loop(0, _ICH // _NL, fill_ones, 0)

    def fill_zero(i, _):
        zb_v[pl.ds(i * _NL, _NL)] = zero
        return 0

    lax.fori_loop(0, _ZCH // _NL, fill_zero, 0)

    for k in range(_TSL // _ZCH):
        pltpu.sync_copy(zb_v, bins_sh.at[pl.ds(sid * _TSL + k * _ZCH, _ZCH)])
    plsc.subcore_barrier()

    for c in range(_BPW // _ICH):
        pltpu.sync_copy(
            ones_v, bins_sh.at[idx_v.at[pl.ds(c * _ICH, _ICH)]], add=True
        )
    plsc.subcore_barrier()

    for k in range(_TSL // _ZCH):
        off = sid * _TSL + k * _ZCH
        pltpu.sync_copy(bins_sh.at[pl.ds(off, _ZCH)], zb_v)
        pltpu.sync_copy(zb_v, hist_hbm.at[cid, pl.ds(off, _ZCH)])


@functools.partial(
    pl.kernel,
    mesh=plsc.VectorSubcoreMesh(core_axis_name="c", subcore_axis_name="s"),
    out_type=jax.ShapeDtypeStruct((_NW, _NL), jnp.float32),
    scratch_types=[
        pltpu.VMEM((_DCH,), jnp.float32),
        pltpu.VMEM((_DCH,), jnp.float32),
        pltpu.VMEM((_DCH,), jnp.float32),
        pltpu.VMEM((_DCH,), jnp.float32),
        pltpu.VMEM((_NL,), jnp.float32),
        pltpu.SemaphoreType.DMA,
        pltpu.SemaphoreType.DMA,
    ],
)
def _sc_dot(hist_hbm, rowsums_hbm, out_hbm, rb0, cb0, rb1, cb1, acc_v, sem0, sem1):
    cid = lax.axis_index("c")
    sid = lax.axis_index("s")
    wid = sid * _NC + cid
    bufs = ((rb0, cb0, sem0), (rb1, cb1, sem1))
    nch = _DSL // _DCH

    def start(ch):
        off = sid * _DSL + ch * _DCH
        r, c, s = bufs[ch % 2]
        return (
            pltpu.async_copy(rowsums_hbm.at[pl.ds(off, _DCH)], r, s),
            pltpu.async_copy(hist_hbm.at[cid, pl.ds(off, _DCH)], c, s),
        )

    handles = [start(0)]
    zero = jnp.zeros((_NL,), jnp.float32)
    accs = (zero, zero, zero, zero)
    for ch in range(nch):
        if ch + 1 < nch:
            handles.append(start(ch + 1))
        h1, h2 = handles[ch]
        h1.wait()
        h2.wait()
        rbuf, cbuf, _ = bufs[ch % 2]

        def body(i, a, rbuf=rbuf, cbuf=cbuf):
            a0, a1, a2, a3 = a
            b = i * 4 * _NL

            def fma(acc, o):
                cnt = cbuf[pl.ds(b + o, _NL)]
                prod = jnp.where(cnt > 0.0, rbuf[pl.ds(b + o, _NL)] * cnt, 0.0)
                return acc + prod

            return (fma(a0, 0), fma(a1, _NL), fma(a2, 2 * _NL), fma(a3, 3 * _NL))

        accs = lax.fori_loop(0, _DCH // (4 * _NL), body, accs)

    a0, a1, a2, a3 = accs
    acc_v[...] = (a0 + a1) + (a2 + a3)
    pltpu.sync_copy(acc_v, out_hbm.at[wid])


def _mlp_body(p_ref, w1_ref, b1_ref, w2_ref, b2_ref, wo_ref, bo_ref, o_ref):
    s = jnp.sum(p_ref[...])
    h1 = jnp.tanh(s * w1_ref[...] + b1_ref[...])
    h2 = jnp.tanh(
        jnp.dot(h1, w2_ref[...], preferred_element_type=jnp.float32) + b2_ref[...]
    )
    o_ref[...] = (
        jnp.dot(h2, wo_ref[...], preferred_element_type=jnp.float32) + bo_ref[...]
    )


def _mlp_call(partials, W1, b1, W2, b2, Wout, bout):
    return pl.pallas_call(
        _mlp_body,
        out_shape=jax.ShapeDtypeStruct((1, bout.shape[-1]), jnp.float32),
    )(partials, W1, b1, W2, b2, Wout, bout)


def kernel(words, table, W1, b1, W2, b2, Wout, bout):
    words_t = words.T.astype(jnp.int32)
    hist = _sc_hist(words_t)
    rowsums = _rowsum_call(table.T)
    partials = _sc_dot(hist, rowsums)
    return _mlp_call(
        partials,
        W1,
        b1.reshape(1, -1),
        W2,
        b2.reshape(1, -1),
        Wout,
        bout.reshape(1, -1),
    )
